# Initial kernel scaffold; baseline (speedup 1.0000x reference)
import jax
import jax.numpy as jnp
from jax.experimental import pallas as pl


def kernel(x, emb_table, W, b):
    raise NotImplementedError("write your pallas kernel here")


# speedup vs baseline: 4.4021x; 4.4021x over previous
import functools

import jax
import jax.numpy as jnp
from jax import lax
from jax.experimental import pallas as pl
from jax.experimental.pallas import tpu as pltpu
from jax.experimental.pallas import tpu_sc as plsc

B, L, V, D = 16384, 200, 8, 16
NTOK = B * L
NPAIR = NTOK // 2

NC, NS = 2, 16
NW = NC * NS
PAIRS_PER_W = NPAIR // NW
G_PAIRS = 1024
SUB = G_PAIRS // 128
NGROUPS = PAIRS_PER_W // G_PAIRS


def _lt_body(emb_ref, wt_ref, b_ref, o_ref):
    o_ref[...] = (
        jnp.dot(emb_ref[...], wt_ref[...], preferred_element_type=jnp.float32)
        + b_ref[...]
    )


def _fused_table(emb_table, W, b):
    return pl.pallas_call(
        _lt_body,
        out_shape=jax.ShapeDtypeStruct((V, V), jnp.float32),
    )(emb_table, W.T, b.reshape(1, V))


def _sc_body(x_hbm, pt_hbm, out_hbm, xbuf, idx2, rows, sem):
    c = lax.axis_index("c")
    s = lax.axis_index("s")
    wid = s * NC + c
    base_w = wid * PAIRS_PER_W
    iota = lax.iota(jnp.int32, 16)
    ev0 = iota * 2
    od0 = ev0 + 1

    def body(g, carry):
        pbase = base_w + g * G_PAIRS
        pltpu.sync_copy(x_hbm.at[pl.ds(pbase * 2, 2 * G_PAIRS)], xbuf)
        for i in range(G_PAIRS // 16):
            ev = plsc.load_gather(xbuf, [ev0 + 32 * i])
            od = plsc.load_gather(xbuf, [od0 + 32 * i])
            idx2[i // 8, pl.ds((i % 8) * 16, 16)] = ev * V + od
        descs = [
            pltpu.async_copy(
                pt_hbm.at[idx2.at[j]], rows.at[pl.ds(j * 128, 128)], sem
            )
            for j in range(SUB)
        ]
        for d in descs:
            d.wait()
        pltpu.sync_copy(rows, out_hbm.at[pl.ds(pbase, G_PAIRS)])
        return carry

    lax.fori_loop(0, NGROUPS, body, 0)


@functools.partial(jax.jit, static_argnames=())
def kernel(x, emb_table, W, b):
    lt = _fused_table(emb_table, W, b)
    pt = jnp.concatenate([jnp.repeat(lt, V, axis=0), jnp.tile(lt, (V, 1))], axis=1)
    x_flat = x.reshape(-1).astype(jnp.int32)

    mesh = plsc.VectorSubcoreMesh(core_axis_name="c", subcore_axis_name="s")
    out2 = pl.kernel(
        _sc_body,
        out_type=jax.ShapeDtypeStruct((NPAIR, D), jnp.float32),
        mesh=mesh,
        compiler_params=pltpu.CompilerParams(
            needs_layout_passes=False, use_tc_tiling_on_sc=False
        ),
        scratch_types=[
            pltpu.VMEM((2 * G_PAIRS,), jnp.int32),
            pltpu.VMEM((SUB, 128), jnp.int32),
            pltpu.VMEM((G_PAIRS, D), jnp.float32),
            pltpu.SemaphoreType.DMA,
        ],
    )(x_flat, pt)
    return out2.reshape(B, L, V)


# speedup vs baseline: 40.4259x; 9.1833x over previous
import functools

import jax
import jax.numpy as jnp
from jax import lax
from jax.experimental import pallas as pl
from jax.experimental.pallas import tpu as pltpu
from jax.experimental.pallas import tpu_sc as plsc

B, L, V, D = 16384, 200, 8, 16

NC, NS = 2, 16
NW = NC * NS
BW = B // NW
LB = 8
NCHUNK = L // LB
NGRP = LB * BW // 16


def _lt_body(emb_ref, wt_ref, b_ref, o_ref):
    o_ref[...] = (
        jnp.dot(emb_ref[...], wt_ref[...], preferred_element_type=jnp.float32)
        + b_ref[...]
    )


def _fused_table(emb_table, W, b):
    return pl.pallas_call(
        _lt_body,
        out_shape=jax.ShapeDtypeStruct((V, V), jnp.float32),
    )(emb_table, W.T, b.reshape(1, V))


def _sc_body(x2_hbm, t_hbm, o_hbm, xchunk, ltbuf, obuf):
    c = lax.axis_index("c")
    s = lax.axis_index("s")
    wid = s * NC + c
    b0 = wid * BW
    pltpu.sync_copy(t_hbm, ltbuf)

    def chunk(ci, carry):
        l0 = ci * LB
        pltpu.sync_copy(x2_hbm.at[pl.ds(l0, LB), pl.ds(b0, BW)], xchunk)

        def grp(g, carry2):
            l = g // (BW // 16)
            bo = (g % (BW // 16)) * 16
            xv = xchunk[l, pl.ds(bo, 16)]
            for v in range(V):
                ov = plsc.load_gather(ltbuf, [xv + (V * v)])
                obuf[l, v, pl.ds(bo, 16)] = ov
            return carry2

        lax.fori_loop(0, NGRP, grp, 0)
        pltpu.sync_copy(obuf, o_hbm.at[pl.ds(l0, LB), :, pl.ds(b0, BW)])
        return carry

    lax.fori_loop(0, NCHUNK, chunk, 0)


@functools.partial(jax.jit, static_argnames=())
def kernel(x, emb_table, W, b):
    lt = _fused_table(emb_table, W, b)
    tflat = jnp.transpose(lt).reshape(V * V)
    x2 = x.astype(jnp.int32).T

    mesh = plsc.VectorSubcoreMesh(core_axis_name="c", subcore_axis_name="s")
    o3 = pl.kernel(
        _sc_body,
        out_type=jax.ShapeDtypeStruct((L, V, B), jnp.float32),
        mesh=mesh,
        compiler_params=pltpu.CompilerParams(needs_layout_passes=False),
        scratch_types=[
            pltpu.VMEM((LB, BW), jnp.int32),
            pltpu.VMEM((V * V,), jnp.float32),
            pltpu.VMEM((LB, V, BW), jnp.float32),
        ],
    )(x2, tflat)
    return jnp.transpose(o3, (2, 0, 1))


# speedup vs baseline: 177.3575x; 4.3872x over previous
import functools

import jax
import jax.numpy as jnp
from jax import lax
from jax.experimental import pallas as pl
from jax.experimental.pallas import tpu as pltpu
from jax.experimental.pallas import tpu_sc as plsc

B, L, V, D = 16384, 200, 8, 16

NC, NS = 2, 16
NW = NC * NS
BW = B // NW
LB = 4
NCHUNK = L // LB
GPL = BW // 16
NGRP = LB * GPL


def _lt_body(emb_ref, wt_ref, b_ref, o_ref):
    o_ref[...] = (
        jnp.dot(emb_ref[...], wt_ref[...], preferred_element_type=jnp.float32)
        + b_ref[...]
    )


def _fused_table(emb_table, W, b):
    return pl.pallas_call(
        _lt_body,
        out_shape=jax.ShapeDtypeStruct((V, V), jnp.float32),
    )(emb_table, W.T, b.reshape(1, V))


def _sc_body(x2_hbm, t_hbm, o_hbm, ltbuf, xA, xB, oA, oB, sxA, sxB, soA, soB):
    c = lax.axis_index("c")
    s = lax.axis_index("s")
    wid = s * NC + c
    b0 = wid * BW
    pltpu.sync_copy(t_hbm, ltbuf)

    def fire_x(ci, xbuf, sem):
        ci = jnp.minimum(ci, NCHUNK - 1)
        pltpu.async_copy(
            x2_hbm.at[pl.ds(ci * LB, LB), pl.ds(b0, BW)], xbuf, sem
        )

    def wait_x(xbuf, sem):
        pltpu.make_async_copy(
            x2_hbm.at[pl.ds(0, LB), pl.ds(b0, BW)], xbuf, sem
        ).wait()

    def wait_o(obuf, sem):
        pltpu.make_async_copy(
            obuf, o_hbm.at[pl.ds(0, LB), :, pl.ds(b0, BW)], sem
        ).wait()

    def compute(xbuf, obuf):
        @plsc.parallel_loop(0, NGRP, unroll=4)
        def grp(g):
            l = g // GPL
            bo = (g % GPL) * 16
            xv = xbuf[l, pl.ds(bo, 16)]
            for v in range(V):
                obuf[l, v, pl.ds(bo, 16)] = plsc.load_gather(
                    ltbuf, [xv + (V * v)]
                )

    def do_chunk(ci, xbuf, obuf, sx, so):
        wait_x(xbuf, sx)
        compute(xbuf, obuf)
        pltpu.async_copy(
            obuf, o_hbm.at[pl.ds(ci * LB, LB), :, pl.ds(b0, BW)], so
        )
        fire_x(ci + 2, xbuf, sx)

    fire_x(jnp.int32(0), xA, sxA)
    fire_x(jnp.int32(1), xB, sxB)
    do_chunk(jnp.int32(0), xA, oA, sxA, soA)
    do_chunk(jnp.int32(1), xB, oB, sxB, soB)

    def body(h, carry):
        wait_o(oA, soA)
        do_chunk(2 * h, xA, oA, sxA, soA)
        wait_o(oB, soB)
        do_chunk(2 * h + 1, xB, oB, sxB, soB)
        return carry

    lax.fori_loop(1, NCHUNK // 2, body, 0)

    wait_o(oA, soA)
    wait_o(oB, soB)
    wait_x(xA, sxA)
    wait_x(xB, sxB)


@functools.partial(jax.jit, static_argnames=())
def kernel(x, emb_table, W, b):
    lt = _fused_table(emb_table, W, b)
    tflat = jnp.transpose(lt).reshape(V * V)
    x2 = x.astype(jnp.int32).T

    mesh = plsc.VectorSubcoreMesh(core_axis_name="c", subcore_axis_name="s")
    o3 = pl.kernel(
        _sc_body,
        out_type=jax.ShapeDtypeStruct((L, V, B), jnp.float32),
        mesh=mesh,
        compiler_params=pltpu.CompilerParams(needs_layout_passes=False),
        scratch_types=[
            pltpu.VMEM((V * V,), jnp.float32),
            pltpu.VMEM((LB, BW), jnp.int32),
            pltpu.VMEM((LB, BW), jnp.int32),
            pltpu.VMEM((LB, V, BW), jnp.float32),
            pltpu.VMEM((LB, V, BW), jnp.float32),
            pltpu.SemaphoreType.DMA,
            pltpu.SemaphoreType.DMA,
            pltpu.SemaphoreType.DMA,
            pltpu.SemaphoreType.DMA,
        ],
    )(x2, tflat)
    return jnp.transpose(o3, (2, 0, 1))
